# baseline (speedup 1.0000x reference)
import functools

import jax
import jax.numpy as jnp
from jax import lax
from jax.experimental import pallas as pl
from jax.experimental.pallas import tpu as pltpu
from jax.experimental.pallas import tpu_sc as plsc

F32 = jnp.float32

_NC, _NS = 2, 16
_NW = _NC * _NS
_DT = 128
_GW = 128
_SW = 128
_CW = 32
_BE = 3200
_BN = 2000
_EPS = 1e-8


def _sc_gather(table, idx):
    t = idx.shape[0]
    nwin = t // _GW
    kmax = -(-nwin // _NW)
    mesh = plsc.VectorSubcoreMesh(core_axis_name="c", subcore_axis_name="s")

    @functools.partial(
        pl.kernel,
        out_type=jax.ShapeDtypeStruct((t, _DT), F32),
        mesh=mesh,
        scratch_types=[
            pltpu.VMEM((1, _GW), jnp.int32),
            pltpu.VMEM((1, _GW, _DT), F32),
            pltpu.SemaphoreType.DMA,
        ],
    )
    def k(table_hbm, idx_hbm, out_hbm, idx_v, rows_v, sem):
        wid = lax.axis_index("s") * _NC + lax.axis_index("c")

        @pl.loop(0, kmax)
        def _(kk):
            w = wid + kk * _NW

            @pl.when(w < nwin)
            def _():
                off = w * _GW
                pltpu.sync_copy(idx_hbm.at[pl.ds(off, _GW)], idx_v.at[0])
                pltpu.async_copy(table_hbm.at[idx_v.at[0]], rows_v.at[0],
                                 sem).wait()
                pltpu.sync_copy(rows_v.at[0], out_hbm.at[pl.ds(off, _GW)])

    return k(table, idx)


def _sc_scatter(dst_idx, mA, mB, zeros):
    n = zeros.shape[0]
    e = dst_idx.shape[0]
    nwin = e // _SW
    half = nwin // 2
    rps = n // _NS
    mesh = plsc.VectorSubcoreMesh(core_axis_name="c", subcore_axis_name="s")
    out = jax.ShapeDtypeStruct((n, _CW), F32)

    @functools.partial(
        pl.kernel,
        out_type=(out,) * 6,
        mesh=mesh,
        scratch_types=[
            pltpu.VMEM_SHARED((n, _CW), F32),
            pltpu.VMEM((1, _SW), jnp.int32),
            pltpu.VMEM((1, _SW, _CW), F32),
            pltpu.SemaphoreType.DMA,
        ],
        compiler_params=pltpu.CompilerParams(use_tc_tiling_on_sc=False),
    )
    def k(dst_hbm, mAh, mBh, z_hbm, o0, o1, o2a, o2b, o3, o4,
          acc, idx_v, upd_v, sem):
        cid = lax.axis_index("c")
        sid = lax.axis_index("s")
        r0 = sid * rps
        tasks = (
            (0, 0, 0, nwin, o0, 0),
            (0, 32, 0, nwin, o1, 0),
            (0, 64, 0, half, o2a, 0),
            (0, 64, half, nwin, o2b, 1),
            (0, 96, 0, nwin, o3, 1),
            (1, 0, 0, nwin, o4, 1),
        )
        for ai, co, wlo, whi, oh, core in tasks:
            mh = (mAh, mBh)[ai]
            kmax = -(-(whi - wlo) // _NS)

            @pl.when(cid == core)
            def _(mh=mh, oh=oh, co=co, wlo=wlo, whi=whi, kmax=kmax):
                pltpu.sync_copy(z_hbm.at[pl.ds(r0, rps)],
                                acc.at[pl.ds(r0, rps)])
                plsc.subcore_barrier()

                @pl.loop(0, kmax)
                def _(kk):
                    w = wlo + sid + kk * _NS

                    @pl.when(w < whi)
                    def _():
                        off = w * _SW
                        pltpu.sync_copy(dst_hbm.at[pl.ds(off, _SW)],
                                        idx_v.at[0])
                        pltpu.sync_copy(
                            mh.at[pl.ds(off, _SW), pl.ds(co, _CW)],
                            upd_v.at[0])
                        pltpu.async_copy(upd_v.at[0], acc.at[idx_v.at[0]],
                                         sem, add=True).wait()

                plsc.subcore_barrier()
                pltpu.sync_copy(acc.at[pl.ds(r0, rps)],
                                oh.at[pl.ds(r0, rps)])
                plsc.subcore_barrier()

    return k(dst_idx, mA, mB, zeros)


def _vec_norm(v3, eps=_EPS):
    return jnp.sqrt(jnp.maximum(v3[0] * v3[0] + v3[1] * v3[1]
                                + v3[2] * v3[2], eps))


def _dot(a, b):
    return jnp.dot(a, b, preferred_element_type=F32)


def _dot16(a, b):
    return jnp.dot(a.astype(jnp.bfloat16), b.astype(jnp.bfloat16),
                   preferred_element_type=F32)


def _gvp_tail(vh, s_pieces, ws_v, b, wv, acts, dot=_dot):
    vn = _vec_norm(vh)
    so = b[0:1, :]
    for piece, w in s_pieces:
        so = so + dot(piece, w)
    so = so + dot(vn, ws_v)
    vo = [dot(vh[x], wv) for x in range(3)]
    if acts:
        nrm = _vec_norm(vo)
        sg = jax.nn.sigmoid(nrm)
        vo = [v * sg for v in vo]
        so = jnp.maximum(so, 0.0)
    return so, vo


def _unpack_v(rows):
    u = lax.bitcast_convert_type(rows[:, 100:124], jnp.uint32)
    lo = lax.bitcast_convert_type(u << 16, F32)
    hi = lax.bitcast_convert_type(u & jnp.uint32(0xFFFF0000), F32)
    return lo, hi


def _vx_slices(lo, hi):
    return [
        lo[:, 0:16],
        jnp.concatenate([lo[:, 16:24], hi[:, 0:8]], axis=-1),
        hi[:, 8:24],
    ]


def _edge_body(srcg, dstg, es, ev0, ev1, ev2,
               wh_s, wh_e, wh_d, ws_s, ws_e, ws_d, ws_v, b0, wv0,
               wh1, ws1_s, ws1_v, b1, wv1,
               wh2, ws2_s, ws2_v, b2, wv2,
               oA, oB):
    src = srcg[...]
    dst = dstg[...]
    e_st = es[...].astype(jnp.bfloat16)
    e_v = [r[...].reshape(-1, 1) for r in (ev0, ev1, ev2)]
    s_src = src[:, :100]
    s_dst = dst[:, :100]
    vs3 = _vx_slices(*_unpack_v(src))
    vd3 = _vx_slices(*_unpack_v(dst))
    vh = []
    for x in range(3):
        vh.append(_dot16(vs3[x], wh_s[...]) + e_v[x] * wh_e[0:1, :]
                  + _dot16(vd3[x], wh_d[...]))
    es_contrib = lax.dot_general(
        e_st, ws_e[...].astype(jnp.bfloat16), (((0,), (0,)), ((), ())),
        preferred_element_type=F32)
    s, vo = _gvp_tail(
        vh,
        [(s_src, ws_s[...]), (s_dst, ws_d[...])],
        ws_v[0:33], b0, wv0[0:33], acts=False, dot=_dot16)
    s = s + es_contrib
    nrm = _vec_norm(vo)
    sg = jax.nn.sigmoid(nrm)
    vo = [v * sg for v in vo]
    s = jnp.maximum(s, 0.0)
    vh = [_dot16(vo[x], wh1[...]) for x in range(3)]
    s, vo = _gvp_tail(vh, [(s, ws1_s[...])], ws1_v[...], b1, wv1[...],
                      acts=True, dot=_dot16)
    vh = [_dot16(vo[x], wh2[...]) for x in range(3)]
    s, vo = _gvp_tail(vh, [(s, ws2_s[...])], ws2_v[...], b2, wv2[...],
                      acts=False, dot=_dot16)
    mv = jnp.concatenate(vo, axis=-1)
    one = jnp.ones_like(s[:, 0:1])
    oA[...] = jnp.concatenate([s, mv[:, 0:27], one], axis=-1)
    zero = jnp.zeros_like(s[:, 0:100])
    oB[...] = jnp.concatenate([mv[:, 27:48], zero, zero[:, 0:7]], axis=-1)


def _layer_norm(s, v, g, b):
    mu = jnp.mean(s, axis=-1, keepdims=True)
    var = jnp.mean((s - mu) * (s - mu), axis=-1, keepdims=True)
    s = (s - mu) / jnp.sqrt(var + 1e-5) * g[0:1, :] + b[0:1, :]
    vx, vy, vz = v[:, 0:16], v[:, 16:32], v[:, 32:48]
    ssq = jnp.maximum(vx * vx + vy * vy + vz * vz, _EPS)
    vnorm = jnp.sqrt(jnp.mean(ssq, axis=-1, keepdims=True))
    return s, v / vnorm


def _node_body(ns, nv, a0, a1, a2a, a2b, a3, a4,
               ln0g, ln0b, ln1g, ln1b,
               f0wh, f0ws_s, f0ws_v, f0b, f0wv,
               f1wh, f1ws_s, f1ws_v, f1b, f1wv,
               os_ref, ov_ref):
    s0 = ns[...]
    v0 = nv[...]
    A0, A1, A3, A4 = a0[...], a1[...], a3[...], a4[...]
    A2 = a2a[...] + a2b[...]
    cnt = A3[:, 31:32]
    inv = 1.0 / jnp.maximum(cnt, 1.0)
    agg_s = jnp.concatenate([A0, A1, A2, A3[:, 0:4]], axis=-1) * inv
    agg_v = jnp.concatenate([A3[:, 4:31], A4[:, 0:21]], axis=-1) * inv
    s, v = _layer_norm(s0 + agg_s, v0 + agg_v, ln0g, ln0b)
    vl = [v[:, 16 * x:16 * (x + 1)] for x in range(3)]
    vh = [_dot(vl[x], f0wh[...]) for x in range(3)]
    fs, fv = _gvp_tail(vh, [(s, f0ws_s[...])], f0ws_v[...], f0b, f0wv[...],
                       acts=True)
    vh = [_dot(fv[x], f1wh[...]) for x in range(3)]
    fs, fv = _gvp_tail(vh, [(fs, f1ws_s[...])], f1ws_v[...], f1b,
                       f1wv[...], acts=False)
    s2 = s + fs
    v2 = jnp.concatenate([vl[x] + fv[x] for x in range(3)], axis=-1)
    s2, v2 = _layer_norm(s2, v2, ln1g, ln1b)
    os_ref[...] = s2
    ov_ref[...] = v2


def _pad8(w):
    r = (-w.shape[0]) % 8
    if r:
        w = jnp.pad(w, ((0, r), (0, 0)))
    return w


def _row(v):
    return _pad8(v.reshape(1, -1))


def kernel(node_s, node_v, edge_index, edge_s, edge_v, params):
    n, si = node_s.shape
    vi = node_v.shape[1]
    e = edge_s.shape[0]

    nvf = node_v.transpose(0, 2, 1).reshape(n, 3 * vi)
    pairs = jnp.stack([nvf[:, 0:24], nvf[:, 24:48]], axis=-1)
    packed = lax.bitcast_convert_type(
        pairs.astype(jnp.bfloat16), F32)
    pad = jnp.zeros((n, _DT - si - 24), F32)
    table = jnp.concatenate([node_s, packed, pad], axis=-1)
    idx = jnp.concatenate([edge_index[0], edge_index[1]])
    evp = [edge_v[:, 0, k].reshape(e // _BE, 1, _BE) for k in range(3)]

    p0, p1, p2 = params['m0'], params['m1'], params['m2']
    whT0 = p0['wh'].T
    wsT0 = p0['ws_w'].T
    w_edge = {
        'wh_s': whT0[0:16], 'wh_e': _pad8(whT0[16:17]), 'wh_d': whT0[17:33],
        'ws_s': wsT0[0:100], 'ws_e': wsT0[100:132], 'ws_d': wsT0[132:232],
        'ws_v': _pad8(wsT0[232:265]),
        'b0': _row(p0['ws_b']), 'wv0': _pad8(p0['wv'].T),
        'wh1': p1['wh'].T, 'ws1_s': p1['ws_w'].T[0:100],
        'ws1_v': p1['ws_w'].T[100:116], 'b1': _row(p1['ws_b']),
        'wv1': p1['wv'].T,
        'wh2': p2['wh'].T, 'ws2_s': p2['ws_w'].T[0:100],
        'ws2_v': p2['ws_w'].T[100:116], 'b2': _row(p2['ws_b']),
        'wv2': p2['wv'].T,
    }
    f0, f1 = params['f0'], params['f1']
    w_node = {
        'ln0g': _row(params['ln0_g']), 'ln0b': _row(params['ln0_b']),
        'ln1g': _row(params['ln1_g']), 'ln1b': _row(params['ln1_b']),
        'f0wh': f0['wh'].T, 'f0ws_s': f0['ws_w'].T[0:100],
        'f0ws_v': f0['ws_w'].T[100:132], 'f0b': _row(f0['ws_b']),
        'f0wv': f0['wv'].T,
        'f1wh': f1['wh'].T, 'f1ws_s': f1['ws_w'].T[0:400],
        'f1ws_v': f1['ws_w'].T[400:432], 'f1b': _row(f1['ws_b']),
        'f1wv': f1['wv'].T,
    }

    g = _sc_gather(table, idx)

    ge = e // _BE
    wspec = [pl.BlockSpec(w.shape, lambda i: (0, 0)) for w in
             w_edge.values()]
    mspec = pl.BlockSpec((_BE, 128), lambda i: (i, 0))
    mA, mB = pl.pallas_call(
        _edge_body,
        grid=(ge,),
        in_specs=[
            pl.BlockSpec((_BE, _DT), lambda i: (i, 0)),
            pl.BlockSpec((_BE, _DT), lambda i, off=ge: (i + off, 0)),
            pl.BlockSpec((32, _BE), lambda i: (0, i)),
            pl.BlockSpec((1, 1, _BE), lambda i: (i, 0, 0)),
            pl.BlockSpec((1, 1, _BE), lambda i: (i, 0, 0)),
            pl.BlockSpec((1, 1, _BE), lambda i: (i, 0, 0)),
        ] + wspec,
        out_specs=[mspec, mspec],
        out_shape=[jax.ShapeDtypeStruct((e, 128), F32)] * 2,
        compiler_params=pltpu.CompilerParams(
            dimension_semantics=("parallel",)),
    )(g, g, edge_s.T, *evp, *w_edge.values())

    zeros = jnp.zeros((n, _CW), F32)
    aggs = _sc_scatter(edge_index[1], mA, mB, zeros)

    gn = n // _BN
    wspec_n = [pl.BlockSpec(w.shape, lambda i: (0, 0)) for w in
               w_node.values()]
    aspec = pl.BlockSpec((_BN, _CW), lambda i: (i, 0))
    s2, v2f = pl.pallas_call(
        _node_body,
        grid=(gn,),
        in_specs=[
            pl.BlockSpec((_BN, si), lambda i: (i, 0)),
            pl.BlockSpec((_BN, 3 * vi), lambda i: (i, 0)),
        ] + [aspec] * 6 + wspec_n,
        out_specs=[
            pl.BlockSpec((_BN, si), lambda i: (i, 0)),
            pl.BlockSpec((_BN, 3 * vi), lambda i: (i, 0)),
        ],
        out_shape=[
            jax.ShapeDtypeStruct((n, si), F32),
            jax.ShapeDtypeStruct((n, 3 * vi), F32),
        ],
        compiler_params=pltpu.CompilerParams(
            dimension_semantics=("parallel",)),
    )(node_s, nvf, *aggs, *w_node.values())

    v2 = v2f.reshape(n, 3, vi).transpose(0, 2, 1)
    return s2, v2


# speedup vs baseline: 1.1568x; 1.1568x over previous
import functools

import jax
import jax.numpy as jnp
from jax import lax
from jax.experimental import pallas as pl
from jax.experimental.pallas import tpu as pltpu
from jax.experimental.pallas import tpu_sc as plsc

F32 = jnp.float32

_NC, _NS = 2, 16
_NW = _NC * _NS
_DT = 128
_GW = 128
_SW = 128
_CW = 32
_BE = 3200
_NSLAB = 5
_BN = 2000
_EPS = 1e-8


def _sc_gather(table, idx):
    t = idx.shape[0]
    nwin = t // _GW
    kmax = -(-nwin // _NW)
    mesh = plsc.VectorSubcoreMesh(core_axis_name="c", subcore_axis_name="s")

    @functools.partial(
        pl.kernel,
        out_type=jax.ShapeDtypeStruct((t, _DT), F32),
        mesh=mesh,
        scratch_types=[
            pltpu.VMEM((1, _GW), jnp.int32),
            pltpu.VMEM((1, _GW, _DT), F32),
            pltpu.SemaphoreType.DMA,
        ],
    )
    def k(table_hbm, idx_hbm, out_hbm, idx_v, rows_v, sem):
        wid = lax.axis_index("s") * _NC + lax.axis_index("c")

        @pl.loop(0, kmax)
        def _(kk):
            w = wid + kk * _NW

            @pl.when(w < nwin)
            def _():
                off = w * _GW
                pltpu.sync_copy(idx_hbm.at[pl.ds(off, _GW)], idx_v.at[0])
                pltpu.async_copy(table_hbm.at[idx_v.at[0]], rows_v.at[0],
                                 sem).wait()
                pltpu.sync_copy(rows_v.at[0], out_hbm.at[pl.ds(off, _GW)])

    return k(table, idx)


def _sc_scatter(dst_idx, mAs, mBs, zeros):
    n = zeros.shape[0]
    e = dst_idx.shape[0]
    ns = len(mAs)
    se = e // ns
    nwin = se // _SW
    split = ns // 2
    rps = n // _NS
    kmax = -(-nwin // _NS)
    mesh = plsc.VectorSubcoreMesh(core_axis_name="c", subcore_axis_name="s")
    out = jax.ShapeDtypeStruct((n, _CW), F32)

    @functools.partial(
        pl.kernel,
        out_type=(out,) * 6,
        mesh=mesh,
        scratch_types=[
            pltpu.VMEM_SHARED((n, _CW), F32),
            pltpu.VMEM((1, _SW), jnp.int32),
            pltpu.VMEM((1, _SW, _CW), F32),
            pltpu.SemaphoreType.DMA,
        ],
        compiler_params=pltpu.CompilerParams(use_tc_tiling_on_sc=False),
    )
    def k(dst_hbm, *refs):
        mAh = refs[0:ns]
        mBh = refs[ns:2 * ns]
        z_hbm = refs[2 * ns]
        o0, o1, o2a, o2b, o3, o4 = refs[2 * ns + 1:2 * ns + 7]
        acc, idx_v, upd_v, sem = refs[2 * ns + 7:]
        cid = lax.axis_index("c")
        sid = lax.axis_index("s")
        r0 = sid * rps
        tasks = (
            (mAh, 0, 0, ns, o0, 0),
            (mAh, 32, 0, ns, o1, 0),
            (mAh, 64, 0, split, o2a, 0),
            (mAh, 64, split, ns, o2b, 1),
            (mAh, 96, 0, ns, o3, 1),
            (mBh, 0, 0, ns, o4, 1),
        )
        for srcs, co, slo, shi, oh, core in tasks:

            @pl.when(cid == core)
            def _(srcs=srcs, oh=oh, co=co, slo=slo, shi=shi):
                pltpu.sync_copy(z_hbm.at[pl.ds(r0, rps)],
                                acc.at[pl.ds(r0, rps)])
                plsc.subcore_barrier()
                for sl in range(slo, shi):
                    mh = srcs[sl]
                    base = sl * se

                    @pl.loop(0, kmax)
                    def _(kk, mh=mh, base=base):
                        w = sid + kk * _NS

                        @pl.when(w < nwin)
                        def _():
                            off = w * _SW
                            pltpu.sync_copy(
                                dst_hbm.at[pl.ds(base + off, _SW)],
                                idx_v.at[0])
                            pltpu.sync_copy(
                                mh.at[pl.ds(off, _SW), pl.ds(co, _CW)],
                                upd_v.at[0])
                            pltpu.async_copy(upd_v.at[0],
                                             acc.at[idx_v.at[0]],
                                             sem, add=True).wait()

                plsc.subcore_barrier()
                pltpu.sync_copy(acc.at[pl.ds(r0, rps)],
                                oh.at[pl.ds(r0, rps)])
                plsc.subcore_barrier()

    return k(dst_idx, *mAs, *mBs, zeros)


def _vec_norm(v3, eps=_EPS):
    return jnp.sqrt(jnp.maximum(v3[0] * v3[0] + v3[1] * v3[1]
                                + v3[2] * v3[2], eps))


def _dot(a, b):
    return jnp.dot(a, b, preferred_element_type=F32)


def _dot16(a, b):
    return jnp.dot(a.astype(jnp.bfloat16), b.astype(jnp.bfloat16),
                   preferred_element_type=F32)


def _gvp_tail(vh, s_pieces, ws_v, b, wv, acts, dot=_dot):
    vn = _vec_norm(vh)
    so = b[0:1, :]
    for piece, w in s_pieces:
        so = so + dot(piece, w)
    so = so + dot(vn, ws_v)
    vo = [dot(vh[x], wv) for x in range(3)]
    if acts:
        nrm = _vec_norm(vo)
        sg = jax.nn.sigmoid(nrm)
        vo = [v * sg for v in vo]
        so = jnp.maximum(so, 0.0)
    return so, vo


def _unpack_v(rows):
    u = lax.bitcast_convert_type(rows[:, 100:124], jnp.uint32)
    lo = lax.bitcast_convert_type(u << 16, F32)
    hi = lax.bitcast_convert_type(u & jnp.uint32(0xFFFF0000), F32)
    return lo, hi


def _vx_slices(lo, hi):
    return [
        lo[:, 0:16],
        jnp.concatenate([lo[:, 16:24], hi[:, 0:8]], axis=-1),
        hi[:, 8:24],
    ]


def _edge_body(srcg, dstg, es, ev0, ev1, ev2,
               wh_s, wh_e, wh_d, ws_s, ws_e, ws_d, ws_v, b0, wv0,
               wh1, ws1_s, ws1_v, b1, wv1,
               wh2, ws2_s, ws2_v, b2, wv2,
               oA, oB):
    src = srcg[...]
    dst = dstg[...]
    e_st = es[...].astype(jnp.bfloat16)
    e_v = [r[...].reshape(-1, 1) for r in (ev0, ev1, ev2)]
    s_src = src[:, :100]
    s_dst = dst[:, :100]
    vs3 = _vx_slices(*_unpack_v(src))
    vd3 = _vx_slices(*_unpack_v(dst))
    vh = []
    for x in range(3):
        vh.append(_dot16(vs3[x], wh_s[...]) + e_v[x] * wh_e[0:1, :]
                  + _dot16(vd3[x], wh_d[...]))
    es_contrib = lax.dot_general(
        e_st, ws_e[...].astype(jnp.bfloat16), (((0,), (0,)), ((), ())),
        preferred_element_type=F32)
    s, vo = _gvp_tail(
        vh,
        [(s_src, ws_s[...]), (s_dst, ws_d[...])],
        ws_v[0:33], b0, wv0[0:33], acts=False, dot=_dot16)
    s = s + es_contrib
    nrm = _vec_norm(vo)
    sg = jax.nn.sigmoid(nrm)
    vo = [v * sg for v in vo]
    s = jnp.maximum(s, 0.0)
    vh = [_dot16(vo[x], wh1[...]) for x in range(3)]
    s, vo = _gvp_tail(vh, [(s, ws1_s[...])], ws1_v[...], b1, wv1[...],
                      acts=True, dot=_dot16)
    vh = [_dot16(vo[x], wh2[...]) for x in range(3)]
    s, vo = _gvp_tail(vh, [(s, ws2_s[...])], ws2_v[...], b2, wv2[...],
                      acts=False, dot=_dot16)
    mv = jnp.concatenate(vo, axis=-1)
    one = jnp.ones_like(s[:, 0:1])
    oA[...] = jnp.concatenate([s, mv[:, 0:27], one], axis=-1)
    zero = jnp.zeros_like(s[:, 0:100])
    oB[...] = jnp.concatenate([mv[:, 27:48], zero, zero[:, 0:7]], axis=-1)


def _layer_norm(s, v, g, b):
    mu = jnp.mean(s, axis=-1, keepdims=True)
    var = jnp.mean((s - mu) * (s - mu), axis=-1, keepdims=True)
    s = (s - mu) / jnp.sqrt(var + 1e-5) * g[0:1, :] + b[0:1, :]
    vx, vy, vz = v[:, 0:16], v[:, 16:32], v[:, 32:48]
    ssq = jnp.maximum(vx * vx + vy * vy + vz * vz, _EPS)
    vnorm = jnp.sqrt(jnp.mean(ssq, axis=-1, keepdims=True))
    return s, v / vnorm


def _node_body(ns, nv, a0, a1, a2a, a2b, a3, a4,
               ln0g, ln0b, ln1g, ln1b,
               f0wh, f0ws_s, f0ws_v, f0b, f0wv,
               f1wh, f1ws_s, f1ws_v, f1b, f1wv,
               os_ref, ov_ref):
    s0 = ns[...]
    v0 = nv[...]
    A0, A1, A3, A4 = a0[...], a1[...], a3[...], a4[...]
    A2 = a2a[...] + a2b[...]
    cnt = A3[:, 31:32]
    inv = 1.0 / jnp.maximum(cnt, 1.0)
    agg_s = jnp.concatenate([A0, A1, A2, A3[:, 0:4]], axis=-1) * inv
    agg_v = jnp.concatenate([A3[:, 4:31], A4[:, 0:21]], axis=-1) * inv
    s, v = _layer_norm(s0 + agg_s, v0 + agg_v, ln0g, ln0b)
    vl = [v[:, 16 * x:16 * (x + 1)] for x in range(3)]
    vh = [_dot(vl[x], f0wh[...]) for x in range(3)]
    fs, fv = _gvp_tail(vh, [(s, f0ws_s[...])], f0ws_v[...], f0b, f0wv[...],
                       acts=True)
    vh = [_dot(fv[x], f1wh[...]) for x in range(3)]
    fs, fv = _gvp_tail(vh, [(fs, f1ws_s[...])], f1ws_v[...], f1b,
                       f1wv[...], acts=False)
    s2 = s + fs
    v2 = jnp.concatenate([vl[x] + fv[x] for x in range(3)], axis=-1)
    s2, v2 = _layer_norm(s2, v2, ln1g, ln1b)
    os_ref[...] = s2
    ov_ref[...] = v2


def _pad8(w):
    r = (-w.shape[0]) % 8
    if r:
        w = jnp.pad(w, ((0, r), (0, 0)))
    return w


def _row(v):
    return _pad8(v.reshape(1, -1))


def kernel(node_s, node_v, edge_index, edge_s, edge_v, params):
    n, si = node_s.shape
    vi = node_v.shape[1]
    e = edge_s.shape[0]

    nvf = node_v.transpose(0, 2, 1).reshape(n, 3 * vi)
    pairs = jnp.stack([nvf[:, 0:24], nvf[:, 24:48]], axis=-1)
    packed = lax.bitcast_convert_type(
        pairs.astype(jnp.bfloat16), F32)
    pad = jnp.zeros((n, _DT - si - 24), F32)
    table = jnp.concatenate([node_s, packed, pad], axis=-1)
    se = e // _NSLAB

    p0, p1, p2 = params['m0'], params['m1'], params['m2']
    whT0 = p0['wh'].T
    wsT0 = p0['ws_w'].T
    w_edge = {
        'wh_s': whT0[0:16], 'wh_e': _pad8(whT0[16:17]), 'wh_d': whT0[17:33],
        'ws_s': wsT0[0:100], 'ws_e': wsT0[100:132], 'ws_d': wsT0[132:232],
        'ws_v': _pad8(wsT0[232:265]),
        'b0': _row(p0['ws_b']), 'wv0': _pad8(p0['wv'].T),
        'wh1': p1['wh'].T, 'ws1_s': p1['ws_w'].T[0:100],
        'ws1_v': p1['ws_w'].T[100:116], 'b1': _row(p1['ws_b']),
        'wv1': p1['wv'].T,
        'wh2': p2['wh'].T, 'ws2_s': p2['ws_w'].T[0:100],
        'ws2_v': p2['ws_w'].T[100:116], 'b2': _row(p2['ws_b']),
        'wv2': p2['wv'].T,
    }
    f0, f1 = params['f0'], params['f1']
    w_node = {
        'ln0g': _row(params['ln0_g']), 'ln0b': _row(params['ln0_b']),
        'ln1g': _row(params['ln1_g']), 'ln1b': _row(params['ln1_b']),
        'f0wh': f0['wh'].T, 'f0ws_s': f0['ws_w'].T[0:100],
        'f0ws_v': f0['ws_w'].T[100:132], 'f0b': _row(f0['ws_b']),
        'f0wv': f0['wv'].T,
        'f1wh': f1['wh'].T, 'f1ws_s': f1['ws_w'].T[0:400],
        'f1ws_v': f1['ws_w'].T[400:432], 'f1b': _row(f1['ws_b']),
        'f1wv': f1['wv'].T,
    }

    ge = se // _BE
    wspec = [pl.BlockSpec(w.shape, lambda i: (0, 0)) for w in
             w_edge.values()]
    mspec = pl.BlockSpec((_BE, 128), lambda i: (i, 0))
    est = edge_s.T
    mAs, mBs = [], []
    for sl in range(_NSLAB):
        a = sl * se
        idx = jnp.concatenate([edge_index[0, a:a + se],
                               edge_index[1, a:a + se]])
        g = _sc_gather(table, idx)
        evp = [edge_v[a:a + se, 0, kk].reshape(ge, 1, _BE)
               for kk in range(3)]
        mA, mB = pl.pallas_call(
            _edge_body,
            grid=(ge,),
            in_specs=[
                pl.BlockSpec((_BE, _DT), lambda i: (i, 0)),
                pl.BlockSpec((_BE, _DT),
                             lambda i, off=ge: (i + off, 0)),
                pl.BlockSpec((32, _BE), lambda i, b=a // _BE: (0, i + b)),
                pl.BlockSpec((1, 1, _BE), lambda i: (i, 0, 0)),
                pl.BlockSpec((1, 1, _BE), lambda i: (i, 0, 0)),
                pl.BlockSpec((1, 1, _BE), lambda i: (i, 0, 0)),
            ] + wspec,
            out_specs=[mspec, mspec],
            out_shape=[jax.ShapeDtypeStruct((se, 128), F32)] * 2,
            compiler_params=pltpu.CompilerParams(
                dimension_semantics=("parallel",)),
        )(g, g, est, *evp, *w_edge.values())
        mAs.append(mA)
        mBs.append(mB)

    zeros = jnp.zeros((n, _CW), F32)
    aggs = _sc_scatter(edge_index[1], mAs, mBs, zeros)

    gn = n // _BN
    wspec_n = [pl.BlockSpec(w.shape, lambda i: (0, 0)) for w in
               w_node.values()]
    aspec = pl.BlockSpec((_BN, _CW), lambda i: (i, 0))
    s2, v2f = pl.pallas_call(
        _node_body,
        grid=(gn,),
        in_specs=[
            pl.BlockSpec((_BN, si), lambda i: (i, 0)),
            pl.BlockSpec((_BN, 3 * vi), lambda i: (i, 0)),
        ] + [aspec] * 6 + wspec_n,
        out_specs=[
            pl.BlockSpec((_BN, si), lambda i: (i, 0)),
            pl.BlockSpec((_BN, 3 * vi), lambda i: (i, 0)),
        ],
        out_shape=[
            jax.ShapeDtypeStruct((n, si), F32),
            jax.ShapeDtypeStruct((n, 3 * vi), F32),
        ],
        compiler_params=pltpu.CompilerParams(
            dimension_semantics=("parallel",)),
    )(node_s, nvf, *aggs, *w_node.values())

    v2 = v2f.reshape(n, 3, vi).transpose(0, 2, 1)
    return s2, v2
